# baseline (speedup 1.0000x reference)
import functools
import math

import jax
import jax.numpy as jnp
from jax import lax
from jax.experimental import pallas as pl
from jax.experimental.pallas import tpu as pltpu
from jax.experimental.pallas import tpu_sc as plsc

_NUM_UNITS = 1024
_K = 64

_TWO_PI_HI = 6.2831854820251465
_TWO_PI_LO = 2.0 * math.pi - 6.2831854820251465
_INV_TWO_PI = 1.0 / (2.0 * math.pi)
_S1 = 9.9998170357e-01
_S3 = -1.6662794909e-01
_S5 = 8.3104314374e-03
_S7 = -1.9286378239e-04
_S9 = 2.1582785572e-06


def _fast_sin(x):
    k = jnp.round(x * _INV_TWO_PI)
    r = (x - k * _TWO_PI_HI) - k * _TWO_PI_LO
    r2 = r * r
    return r * (_S1 + r2 * (_S3 + r2 * (_S5 + r2 * (_S7 + r2 * _S9))))


def _table_kernel(out_ref, sa_ref, ca_ref, sb_ref, cb_ref, *, t_tile, k,
                  num_units, n_hi):
    t = pl.program_id(0)
    half_pi = jnp.float32(math.pi / 2.0)
    neg_log_rate = jnp.float32(-2.0 * math.log(10000.0) / num_units)
    scale = jnp.float32(num_units**0.5)

    @pl.when(t == 0)
    def _init_tables():
        col8 = jax.lax.broadcasted_iota(jnp.int32, (8, num_units), 1)
        w_row = jnp.exp(col8.astype(jnp.float32) * neg_log_rate)[0:1, :]
        parity8 = (col8 & 1).astype(jnp.float32)
        phase = (parity8 * half_pi)[0:1, :]
        lo = jax.lax.broadcasted_iota(jnp.int32, (k, num_units), 0)
        b = lo.astype(jnp.float32) * w_row + phase
        sb_ref[...] = _fast_sin(b) * scale
        cb_ref[...] = _fast_sin(b + half_pi) * scale
        hi = jax.lax.broadcasted_iota(jnp.int32, (n_hi, num_units), 0)
        a = (hi * k).astype(jnp.float32) * w_row
        sa_ref[...] = _fast_sin(a)
        ca_ref[...] = _fast_sin(a + half_pi)

    chunks = t_tile // k
    for j in range(chunks):
        hi_idx = t * chunks + j
        a_s = sa_ref[pl.ds(hi_idx, 1), :]
        a_c = ca_ref[pl.ds(hi_idx, 1), :]
        val = a_s * cb_ref[...] + a_c * sb_ref[...]
        out_ref[j * k:(j + 1) * k, :] = val

    @pl.when(t == 0)
    def _zero_row0():
        out_ref[0:1, :] = jnp.zeros((1, num_units), jnp.float32)


def _make_table(t_total, num_units):
    t_tile = 256
    k = _K
    n_hi = t_total // k
    return pl.pallas_call(
        functools.partial(_table_kernel, t_tile=t_tile, k=k,
                          num_units=num_units, n_hi=n_hi),
        grid=(t_total // t_tile,),
        out_specs=pl.BlockSpec((t_tile, num_units), lambda t: (t, 0)),
        out_shape=jax.ShapeDtypeStruct((t_total, num_units), jnp.float32),
        scratch_shapes=[
            pltpu.VMEM((n_hi, num_units), jnp.float32),
            pltpu.VMEM((n_hi, num_units), jnp.float32),
            pltpu.VMEM((k, num_units), jnp.float32),
            pltpu.VMEM((k, num_units), jnp.float32),
        ],
    )()


def kernel(inputs):
    n, t_total = inputs.shape
    num_units = _NUM_UNITS
    table = _make_table(t_total, num_units)

    info = plsc.get_sparse_core_info()
    nw = info.num_cores * info.num_subcores
    rows_per_w = t_total // nw
    mesh = plsc.VectorSubcoreMesh(core_axis_name="c", subcore_axis_name="s")

    @functools.partial(
        pl.kernel, mesh=mesh,
        out_type=jax.ShapeDtypeStruct((n, t_total, num_units), jnp.float32),
    )
    def _broadcast_sc(table_hbm, out_hbm):
        wid = lax.axis_index("s") * info.num_cores + lax.axis_index("c")
        base = wid * rows_per_w
        src = table_hbm.at[pl.ds(base, rows_per_w), :]
        for n_i in range(n):
            pltpu.sync_copy(src, out_hbm.at[n_i, pl.ds(base, rows_per_w), :])

    return _broadcast_sc(table)


# speedup vs baseline: 1.0004x; 1.0004x over previous
import functools
import math

import jax
import jax.numpy as jnp
from jax import lax
from jax.experimental import pallas as pl
from jax.experimental.pallas import tpu as pltpu
from jax.experimental.pallas import tpu_sc as plsc

_NUM_UNITS = 1024
_K = 64

_TWO_PI_HI = 6.2831854820251465
_TWO_PI_LO = 2.0 * math.pi - 6.2831854820251465
_INV_TWO_PI = 1.0 / (2.0 * math.pi)
_S1 = 9.9998170357e-01
_S3 = -1.6662794909e-01
_S5 = 8.3104314374e-03
_S7 = -1.9286378239e-04
_S9 = 2.1582785572e-06


def _fast_sin(x):
    k = jnp.round(x * _INV_TWO_PI)
    r = (x - k * _TWO_PI_HI) - k * _TWO_PI_LO
    r2 = r * r
    return r * (_S1 + r2 * (_S3 + r2 * (_S5 + r2 * (_S7 + r2 * _S9))))


def _table_kernel(out_ref, sa_ref, ca_ref, sb_ref, cb_ref, *, t_tile, k,
                  num_units, n_hi):
    t = pl.program_id(0)
    half_pi = jnp.float32(math.pi / 2.0)
    neg_log_rate = jnp.float32(-2.0 * math.log(10000.0) / num_units)
    scale = jnp.float32(num_units**0.5)

    @pl.when(t == 0)
    def _init_tables():
        col8 = jax.lax.broadcasted_iota(jnp.int32, (8, num_units), 1)
        w_row = jnp.exp(col8.astype(jnp.float32) * neg_log_rate)[0:1, :]
        parity8 = (col8 & 1).astype(jnp.float32)
        phase = (parity8 * half_pi)[0:1, :]
        lo = jax.lax.broadcasted_iota(jnp.int32, (k, num_units), 0)
        b = lo.astype(jnp.float32) * w_row + phase
        sb_ref[...] = _fast_sin(b) * scale
        cb_ref[...] = _fast_sin(b + half_pi) * scale
        hi = jax.lax.broadcasted_iota(jnp.int32, (n_hi, num_units), 0)
        a = (hi * k).astype(jnp.float32) * w_row
        sa_ref[...] = _fast_sin(a)
        ca_ref[...] = _fast_sin(a + half_pi)

    chunks = t_tile // k
    for j in range(chunks):
        hi_idx = t * chunks + j
        a_s = sa_ref[pl.ds(hi_idx, 1), :]
        a_c = ca_ref[pl.ds(hi_idx, 1), :]
        val = a_s * cb_ref[...] + a_c * sb_ref[...]
        out_ref[j * k:(j + 1) * k, :] = val

    @pl.when(t == 0)
    def _zero_row0():
        out_ref[0:1, :] = jnp.zeros((1, num_units), jnp.float32)


def _make_table(t_total, num_units):
    t_tile = 256
    k = _K
    n_hi = t_total // k
    return pl.pallas_call(
        functools.partial(_table_kernel, t_tile=t_tile, k=k,
                          num_units=num_units, n_hi=n_hi),
        grid=(t_total // t_tile,),
        out_specs=pl.BlockSpec((t_tile, num_units), lambda t: (t, 0)),
        out_shape=jax.ShapeDtypeStruct((t_total, num_units), jnp.float32),
        scratch_shapes=[
            pltpu.VMEM((n_hi, num_units), jnp.float32),
            pltpu.VMEM((n_hi, num_units), jnp.float32),
            pltpu.VMEM((k, num_units), jnp.float32),
            pltpu.VMEM((k, num_units), jnp.float32),
        ],
    )()


def kernel(inputs):
    n, t_total = inputs.shape
    num_units = _NUM_UNITS
    table = _make_table(t_total, num_units)

    info = plsc.get_sparse_core_info()
    nw = info.num_cores * info.num_subcores
    rows_per_w = t_total // nw
    mesh = plsc.VectorSubcoreMesh(core_axis_name="c", subcore_axis_name="s")

    @functools.partial(
        pl.kernel, mesh=mesh,
        out_type=jax.ShapeDtypeStruct((n, t_total, num_units), jnp.float32),
        scratch_types=[pltpu.SemaphoreType.DMA],
    )
    def _broadcast_sc(table_hbm, out_hbm, sem):
        wid = lax.axis_index("s") * info.num_cores + lax.axis_index("c")
        base = wid * rows_per_w
        src = table_hbm.at[pl.ds(base, rows_per_w), :]
        copies = [
            pltpu.async_copy(src, out_hbm.at[n_i, pl.ds(base, rows_per_w), :],
                             sem)
            for n_i in range(n)
        ]
        for c in copies:
            c.wait()

    return _broadcast_sc(table)


# speedup vs baseline: 38.5992x; 38.5852x over previous
import functools
import math

import jax
import jax.numpy as jnp
from jax import lax
from jax.experimental import pallas as pl
from jax.experimental.pallas import tpu as pltpu
from jax.experimental.pallas import tpu_sc as plsc

_NUM_UNITS = 1024
_K = 64

_TWO_PI_HI = 6.2831854820251465
_TWO_PI_LO = 2.0 * math.pi - 6.2831854820251465
_INV_TWO_PI = 1.0 / (2.0 * math.pi)
_S1 = 9.9998170357e-01
_S3 = -1.6662794909e-01
_S5 = 8.3104314374e-03
_S7 = -1.9286378239e-04
_S9 = 2.1582785572e-06


def _fast_sin(x):
    k = jnp.round(x * _INV_TWO_PI)
    r = (x - k * _TWO_PI_HI) - k * _TWO_PI_LO
    r2 = r * r
    return r * (_S1 + r2 * (_S3 + r2 * (_S5 + r2 * (_S7 + r2 * _S9))))


def _table_kernel(out_ref, sa_ref, ca_ref, sb_ref, cb_ref, *, t_tile, k,
                  num_units, n_hi):
    t = pl.program_id(0)
    half_pi = jnp.float32(math.pi / 2.0)
    neg_log_rate = jnp.float32(-2.0 * math.log(10000.0) / num_units)
    scale = jnp.float32(num_units**0.5)

    @pl.when(t == 0)
    def _init_tables():
        col8 = jax.lax.broadcasted_iota(jnp.int32, (8, num_units), 1)
        w_row = jnp.exp(col8.astype(jnp.float32) * neg_log_rate)[0:1, :]
        parity8 = (col8 & 1).astype(jnp.float32)
        phase = (parity8 * half_pi)[0:1, :]
        lo = jax.lax.broadcasted_iota(jnp.int32, (k, num_units), 0)
        b = lo.astype(jnp.float32) * w_row + phase
        sb_ref[...] = _fast_sin(b) * scale
        cb_ref[...] = _fast_sin(b + half_pi) * scale
        hi = jax.lax.broadcasted_iota(jnp.int32, (n_hi, num_units), 0)
        a = (hi * k).astype(jnp.float32) * w_row
        sa_ref[...] = _fast_sin(a)
        ca_ref[...] = _fast_sin(a + half_pi)

    chunks = t_tile // k
    for j in range(chunks):
        hi_idx = t * chunks + j
        a_s = sa_ref[pl.ds(hi_idx, 1), :]
        a_c = ca_ref[pl.ds(hi_idx, 1), :]
        val = a_s * cb_ref[...] + a_c * sb_ref[...]
        out_ref[j * k:(j + 1) * k, :] = val

    @pl.when(t == 0)
    def _zero_row0():
        out_ref[0:1, :] = jnp.zeros((1, num_units), jnp.float32)


def _make_table(t_total, num_units):
    t_tile = 256
    k = _K
    n_hi = t_total // k
    return pl.pallas_call(
        functools.partial(_table_kernel, t_tile=t_tile, k=k,
                          num_units=num_units, n_hi=n_hi),
        grid=(t_total // t_tile,),
        out_specs=pl.BlockSpec((t_tile, num_units), lambda t: (t, 0)),
        out_shape=jax.ShapeDtypeStruct((t_total, num_units), jnp.float32),
        scratch_shapes=[
            pltpu.VMEM((n_hi, num_units), jnp.float32),
            pltpu.VMEM((n_hi, num_units), jnp.float32),
            pltpu.VMEM((k, num_units), jnp.float32),
            pltpu.VMEM((k, num_units), jnp.float32),
        ],
    )()


def kernel(inputs):
    n, t_total = inputs.shape
    num_units = _NUM_UNITS
    table = _make_table(t_total, num_units)

    info = plsc.get_sparse_core_info()
    nw = info.num_cores * info.num_subcores
    rows_per_w = t_total // nw
    mesh = plsc.VectorSubcoreMesh(core_axis_name="c", subcore_axis_name="s")

    chunk = 64
    n_chunks = rows_per_w // chunk

    @functools.partial(
        pl.kernel, mesh=mesh,
        out_type=jax.ShapeDtypeStruct((n, t_total, num_units), jnp.float32),
        scratch_types=[
            pltpu.VMEM((chunk, num_units), jnp.float32),
            pltpu.SemaphoreType.DMA,
        ],
    )
    def _broadcast_sc(table_hbm, out_hbm, buf, sem):
        wid = lax.axis_index("s") * info.num_cores + lax.axis_index("c")
        base = wid * rows_per_w
        for c_i in range(n_chunks):
            row0 = base + c_i * chunk
            pltpu.sync_copy(table_hbm.at[pl.ds(row0, chunk), :], buf)
            copies = [
                pltpu.async_copy(buf, out_hbm.at[n_i, pl.ds(row0, chunk), :],
                                 sem)
                for n_i in range(n)
            ]
            for c in copies:
                c.wait()

    return _broadcast_sc(table)


# speedup vs baseline: 95.2006x; 2.4664x over previous
import functools
import math

import jax
import jax.numpy as jnp
from jax.experimental import pallas as pl
from jax.experimental.pallas import tpu as pltpu

_NUM_UNITS = 1024
_K = 64

_TWO_PI_HI = 6.2831854820251465
_TWO_PI_LO = 2.0 * math.pi - 6.2831854820251465
_INV_TWO_PI = 1.0 / (2.0 * math.pi)
_S1 = 9.9998170357e-01
_S3 = -1.6662794909e-01
_S5 = 8.3104314374e-03
_S7 = -1.9286378239e-04
_S9 = 2.1582785572e-06


def _fast_sin(x):
    k = jnp.round(x * _INV_TWO_PI)
    r = (x - k * _TWO_PI_HI) - k * _TWO_PI_LO
    r2 = r * r
    return r * (_S1 + r2 * (_S3 + r2 * (_S5 + r2 * (_S7 + r2 * _S9))))


def _pe_tile_kernel(out_ref, sa_ref, ca_ref, sb_ref, cb_ref, *, n, t_tile, k,
                    num_units, n_hi):
    t = pl.program_id(0)
    half_pi = jnp.float32(math.pi / 2.0)
    neg_log_rate = jnp.float32(-2.0 * math.log(10000.0) / num_units)
    scale = jnp.float32(num_units**0.5)

    @pl.when(t == 0)
    def _init_tables():
        col8 = jax.lax.broadcasted_iota(jnp.int32, (8, num_units), 1)
        w_row = jnp.exp(col8.astype(jnp.float32) * neg_log_rate)[0:1, :]
        parity8 = (col8 & 1).astype(jnp.float32)
        phase = (parity8 * half_pi)[0:1, :]
        lo = jax.lax.broadcasted_iota(jnp.int32, (k, num_units), 0)
        b = lo.astype(jnp.float32) * w_row + phase
        sb_ref[...] = _fast_sin(b) * scale
        cb_ref[...] = _fast_sin(b + half_pi) * scale
        hi = jax.lax.broadcasted_iota(jnp.int32, (n_hi, num_units), 0)
        a = (hi * k).astype(jnp.float32) * w_row
        sa_ref[...] = _fast_sin(a)
        ca_ref[...] = _fast_sin(a + half_pi)

    chunks = t_tile // k
    for j in range(chunks):
        hi_idx = t * chunks + j
        a_s = sa_ref[pl.ds(hi_idx, 1), :]
        a_c = ca_ref[pl.ds(hi_idx, 1), :]
        val = a_s * cb_ref[...] + a_c * sb_ref[...]
        out_ref[:, j * k:(j + 1) * k, :] = jnp.broadcast_to(
            val[None], (n, k, num_units))

    @pl.when(t == 0)
    def _zero_row0():
        out_ref[:, 0:1, :] = jnp.zeros((n, 1, num_units), jnp.float32)


def kernel(inputs):
    n, t_total = inputs.shape
    num_units = _NUM_UNITS
    t_tile = 256
    k = _K
    n_hi = t_total // k
    grid = (t_total // t_tile,)
    out = pl.pallas_call(
        functools.partial(_pe_tile_kernel, n=n, t_tile=t_tile, k=k,
                          num_units=num_units, n_hi=n_hi),
        grid=grid,
        out_specs=pl.BlockSpec((n, t_tile, num_units), lambda t: (0, t, 0)),
        out_shape=jax.ShapeDtypeStruct((n, t_total, num_units), jnp.float32),
        scratch_shapes=[
            pltpu.VMEM((n_hi, num_units), jnp.float32),
            pltpu.VMEM((n_hi, num_units), jnp.float32),
            pltpu.VMEM((k, num_units), jnp.float32),
            pltpu.VMEM((k, num_units), jnp.float32),
        ],
    )()
    return out
